# baseline (speedup 1.0000x reference)
import functools

import jax
import jax.numpy as jnp
from jax import lax
from jax.experimental import pallas as pl
from jax.experimental.pallas import tpu as pltpu
from jax.experimental.pallas import tpu_sc as plsc

CHUNK = 128


@functools.lru_cache(maxsize=None)
def _build(B, DE, DT, V):
    info = plsc.get_sparse_core_info()
    NC, NS = info.num_cores, info.num_subcores
    NW = NC * NS
    per_w = B // NW
    assert per_w * NW == B and per_w % CHUNK == 0
    k = per_w // CHUNK
    DO = DE + DT

    mesh = plsc.VectorSubcoreMesh(core_axis_name="c", subcore_axis_name="s")

    @functools.partial(
        pl.kernel,
        mesh=mesh,
        out_type=jax.ShapeDtypeStruct((B, DO), jnp.float32),
        scratch_types=[
            pltpu.VMEM((k, CHUNK), jnp.int32),
            pltpu.VMEM((CHUNK, DT), jnp.float32),
            pltpu.SemaphoreType.DMA,
        ],
        compiler_params=pltpu.CompilerParams(use_tc_tiling_on_sc=False),
    )
    def sc_kernel(elmo_hbm, idx_hbm, table_hbm, out_hbm, idx_v, rows_v, sem):
        wid = lax.axis_index("s") * NC + lax.axis_index("c")
        base = wid * per_w
        pltpu.sync_copy(idx_hbm.at[wid], idx_v)
        pltpu.sync_copy(
            elmo_hbm.at[pl.ds(base, per_w)],
            out_hbm.at[pl.ds(base, per_w), pl.ds(0, DE)],
        )

        def body(j, _):
            pltpu.async_copy(table_hbm.at[idx_v.at[j]], rows_v, sem).wait()
            pltpu.sync_copy(
                rows_v,
                out_hbm.at[pl.ds(base + j * CHUNK, CHUNK), pl.ds(DE, DT)],
            )
            return ()

        lax.fori_loop(0, k, body, (), unroll=False)

    return sc_kernel, NW, k


def kernel(elmo_emb, inp, table):
    S0, S1, DE = elmo_emb.shape
    V, DT = table.shape
    B = S0 * S1
    elmo2 = elmo_emb.reshape(B, DE)
    sc_kernel, NW, k = _build(B, DE, DT, V)
    idx = inp.reshape(NW, k, CHUNK).astype(jnp.int32)
    out = sc_kernel(elmo2, idx, table)
    return out.reshape(S0, S1, DE + DT)


# speedup vs baseline: 5.6131x; 5.6131x over previous
import functools

import jax
import jax.numpy as jnp
from jax import lax
from jax.experimental import pallas as pl
from jax.experimental.pallas import tpu as pltpu
from jax.experimental.pallas import tpu_sc as plsc

CHUNK = 128


@functools.lru_cache(maxsize=None)
def _build_gather(B, DT, V):
    info = plsc.get_sparse_core_info()
    NC, NS = info.num_cores, info.num_subcores
    NW = NC * NS
    per_w = B // NW
    assert per_w * NW == B and per_w % CHUNK == 0
    k = per_w // CHUNK
    assert k % 2 == 0
    m = k // 2 - 1

    mesh = plsc.VectorSubcoreMesh(core_axis_name="c", subcore_axis_name="s")

    @functools.partial(
        pl.kernel,
        mesh=mesh,
        out_type=jax.ShapeDtypeStruct((B, DT), jnp.float32),
        scratch_types=[
            pltpu.VMEM((k, CHUNK), jnp.int32),
            pltpu.VMEM((CHUNK, DT), jnp.float32),
            pltpu.VMEM((CHUNK, DT), jnp.float32),
            pltpu.SemaphoreType.DMA,
            pltpu.SemaphoreType.DMA,
            pltpu.SemaphoreType.DMA,
            pltpu.SemaphoreType.DMA,
        ],
        compiler_params=pltpu.CompilerParams(use_tc_tiling_on_sc=False),
    )
    def sc_kernel(idx_hbm, table_hbm, g_hbm, idx_v, buf0, buf1, si0, si1, so0, so1):
        wid = lax.axis_index("s") * NC + lax.axis_index("c")
        base = wid * per_w
        pltpu.sync_copy(idx_hbm.at[wid], idx_v)

        def start_in(j, buf, sem):
            pltpu.async_copy(table_hbm.at[idx_v.at[j]], buf, sem)

        def start_out(j, buf, sem):
            pltpu.async_copy(buf, g_hbm.at[pl.ds(base + j * CHUNK, CHUNK)], sem)

        def wait_in(sem, buf):
            pltpu.make_async_copy(table_hbm.at[idx_v.at[0]], buf, sem).wait()

        def wait_out(sem, buf):
            pltpu.make_async_copy(buf, g_hbm.at[pl.ds(base, CHUNK)], sem).wait()

        start_in(0, buf0, si0)
        start_in(1, buf1, si1)

        def body(i, _):
            j = 2 * i
            wait_in(si0, buf0)
            start_out(j, buf0, so0)
            wait_in(si1, buf1)
            start_out(j + 1, buf1, so1)
            wait_out(so0, buf0)
            start_in(j + 2, buf0, si0)
            wait_out(so1, buf1)
            start_in(j + 3, buf1, si1)
            return ()

        lax.fori_loop(0, m, body, (), unroll=False)

        wait_in(si0, buf0)
        start_out(k - 2, buf0, so0)
        wait_in(si1, buf1)
        start_out(k - 1, buf1, so1)
        wait_out(so0, buf0)
        wait_out(so1, buf1)

    return sc_kernel, NW, k


@functools.lru_cache(maxsize=None)
def _build_concat(S0, S1, DE, DT, BS):
    DO = DE + DT

    def concat_body(elmo_ref, g_ref, out_ref):
        out_ref[...] = jnp.concatenate((elmo_ref[...], g_ref[...]), axis=-1)

    return pl.pallas_call(
        concat_body,
        grid=(S0 // BS,),
        in_specs=[
            pl.BlockSpec((BS, S1, DE), lambda i: (i, 0, 0)),
            pl.BlockSpec((BS, S1, DT), lambda i: (i, 0, 0)),
        ],
        out_specs=pl.BlockSpec((BS, S1, DO), lambda i: (i, 0, 0)),
        out_shape=jax.ShapeDtypeStruct((S0, S1, DO), jnp.float32),
    )


def kernel(elmo_emb, inp, table):
    S0, S1, DE = elmo_emb.shape
    V, DT = table.shape
    B = S0 * S1
    sc_gather, NW, k = _build_gather(B, DT, V)
    idx = inp.reshape(NW, k, CHUNK).astype(jnp.int32)
    g = sc_gather(idx, table)
    concat = _build_concat(S0, S1, DE, DT, 64)
    return concat(elmo_emb, g.reshape(S0, S1, DT))


# speedup vs baseline: 5.9407x; 1.0584x over previous
import functools

import jax
import jax.numpy as jnp
from jax import lax
from jax.experimental import pallas as pl
from jax.experimental.pallas import tpu as pltpu
from jax.experimental.pallas import tpu_sc as plsc

CHUNK = 128
HALF = CHUNK // 2


@functools.lru_cache(maxsize=None)
def _build_gather(B, DT, V):
    info = plsc.get_sparse_core_info()
    NC, NS = info.num_cores, info.num_subcores
    NW = NC * NS
    per_w = B // NW
    assert per_w * NW == B and per_w % CHUNK == 0
    k = per_w // CHUNK
    assert k % 2 == 0
    m = k // 2 - 1

    mesh = plsc.VectorSubcoreMesh(core_axis_name="c", subcore_axis_name="s")

    @functools.partial(
        pl.kernel,
        mesh=mesh,
        out_type=jax.ShapeDtypeStruct((B // 2, 2 * DT), jnp.float32),
        scratch_types=[
            pltpu.VMEM((k, 2, HALF), jnp.int32),
            pltpu.VMEM((2, HALF, DT), jnp.float32),
            pltpu.VMEM((2, HALF, DT), jnp.float32),
            pltpu.SemaphoreType.DMA,
            pltpu.SemaphoreType.DMA,
            pltpu.SemaphoreType.DMA,
            pltpu.SemaphoreType.DMA,
        ],
        compiler_params=pltpu.CompilerParams(use_tc_tiling_on_sc=False),
    )
    def sc_kernel(idx_hbm, table_hbm, g_hbm, idx_v, buf0, buf1, si0, si1, so0, so1):
        wid = lax.axis_index("s") * NC + lax.axis_index("c")
        base = wid * per_w // 2
        pltpu.sync_copy(idx_hbm.at[wid], idx_v)

        def start_in(j, buf, sem):
            pltpu.async_copy(table_hbm.at[idx_v.at[j, 0]], buf.at[0], sem)
            pltpu.async_copy(table_hbm.at[idx_v.at[j, 1]], buf.at[1], sem)

        def start_out(j, buf, sem):
            rows = pl.ds(base + j * HALF, HALF)
            pltpu.async_copy(buf.at[0], g_hbm.at[rows, pl.ds(0, DT)], sem)
            pltpu.async_copy(buf.at[1], g_hbm.at[rows, pl.ds(DT, DT)], sem)

        def wait_in(sem, buf):
            for h in (0, 1):
                pltpu.make_async_copy(
                    table_hbm.at[idx_v.at[0, 0]], buf.at[h], sem
                ).wait()

        def wait_out(sem, buf):
            for h in (0, 1):
                pltpu.make_async_copy(
                    buf.at[h], g_hbm.at[pl.ds(base, HALF), pl.ds(0, DT)], sem
                ).wait()

        start_in(0, buf0, si0)
        start_in(1, buf1, si1)

        def body(i, _):
            j = 2 * i
            wait_in(si0, buf0)
            start_out(j, buf0, so0)
            wait_in(si1, buf1)
            start_out(j + 1, buf1, so1)
            wait_out(so0, buf0)
            start_in(j + 2, buf0, si0)
            wait_out(so1, buf1)
            start_in(j + 3, buf1, si1)
            return ()

        lax.fori_loop(0, m, body, (), unroll=False)

        wait_in(si0, buf0)
        start_out(k - 2, buf0, so0)
        wait_in(si1, buf1)
        start_out(k - 1, buf1, so1)
        wait_out(so0, buf0)
        wait_out(so1, buf1)

    return sc_kernel, NW, k


@functools.lru_cache(maxsize=None)
def _build_concat(S0, S1, DE, DT, BS):
    DO = DE + DT
    grows_blk = BS * S1 // 2

    def concat_body(elmo_ref, g_ref, out_ref):
        g = g_ref[...]
        even = g[:, :DT].reshape(BS, S1 // 2, 1, DT)
        odd = g[:, DT:].reshape(BS, S1 // 2, 1, DT)
        x2 = jnp.concatenate((even, odd), axis=2).reshape(BS, S1, DT)
        out_ref[...] = jnp.concatenate((elmo_ref[...], x2), axis=-1)

    return pl.pallas_call(
        concat_body,
        grid=(S0 // BS,),
        in_specs=[
            pl.BlockSpec((BS, S1, DE), lambda i: (i, 0, 0)),
            pl.BlockSpec((grows_blk, 2 * DT), lambda i: (i, 0)),
        ],
        out_specs=pl.BlockSpec((BS, S1, DO), lambda i: (i, 0, 0)),
        out_shape=jax.ShapeDtypeStruct((S0, S1, DO), jnp.float32),
    )


def kernel(elmo_emb, inp, table):
    S0, S1, DE = elmo_emb.shape
    V, DT = table.shape
    B = S0 * S1
    sc_gather, NW, k = _build_gather(B, DT, V)
    idx = (
        inp.reshape(NW, k, HALF, 2)
        .transpose(0, 1, 3, 2)
        .astype(jnp.int32)
    )
    g = sc_gather(idx, table)
    concat = _build_concat(S0, S1, DE, DT, 64)
    return concat(elmo_emb, g)
